# baseline (speedup 1.0000x reference)
import functools

import jax
import jax.numpy as jnp
from jax import lax
from jax.experimental import pallas as pl
from jax.experimental.pallas import tpu as pltpu
from jax.experimental.pallas import tpu_sc as plsc

EMBED = 64
BATCH = 16384
L = 16
NC, NS = 2, 16
NW = NC * NS
BPW = BATCH // NW

_mesh = plsc.VectorSubcoreMesh(core_axis_name="c", subcore_axis_name="s")


@functools.partial(
    pl.kernel,
    out_type=jax.ShapeDtypeStruct((BATCH,), jnp.float32),
    mesh=_mesh,
    compiler_params=pltpu.CompilerParams(
        use_tc_tiling_on_sc=False, needs_layout_passes=False
    ),
    scratch_types=[
        pltpu.VMEM((BPW,), jnp.int32),
        pltpu.VMEM((BPW,), jnp.int32),
        pltpu.VMEM((BPW, EMBED), jnp.float32),
        pltpu.VMEM((BPW, EMBED), jnp.float32),
        pltpu.VMEM((BPW,), jnp.float32),
        pltpu.SemaphoreType.DMA,
        pltpu.SemaphoreType.DMA,
    ],
)
def _skipgram_sc(cw_hbm, tw_hbm, ct_hbm, tt_hbm, out_hbm,
                 cidx, tidx, crows, trows, outv, sem_c, sem_t):
    wid = lax.axis_index("s") * NC + lax.axis_index("c")
    base = wid * BPW

    pltpu.sync_copy(cw_hbm.at[pl.ds(base, BPW)], cidx)
    pltpu.sync_copy(tw_hbm.at[pl.ds(base, BPW)], tidx)

    cp_c = pltpu.async_copy(ct_hbm.at[cidx], crows, sem_c)
    cp_t = pltpu.async_copy(tt_hbm.at[tidx], trows, sem_t)
    cp_c.wait()
    cp_t.wait()

    lane = lax.iota(jnp.int32, L)

    def group_body(g, carry):
        rows = lane + g * L

        def col_body(j, acc):
            cols = jnp.zeros((L,), jnp.int32) + j
            cv = plsc.load_gather(crows, [rows, cols])
            tv = plsc.load_gather(trows, [rows, cols])
            return acc + cv * tv

        acc = lax.fori_loop(0, EMBED, col_body, jnp.zeros((L,), jnp.float32))
        outv[pl.ds(g * L, L)] = acc
        return carry

    lax.fori_loop(0, BPW // L, group_body, 0)

    pltpu.sync_copy(outv, out_hbm.at[pl.ds(base, BPW)])


def kernel(center_words, target_words, center_table, target_table):
    return _skipgram_sc(
        center_words.astype(jnp.int32),
        target_words.astype(jnp.int32),
        center_table,
        target_table,
    )


# speedup vs baseline: 2.0649x; 2.0649x over previous
import functools

import jax
import jax.numpy as jnp
from jax import lax
from jax.experimental import pallas as pl
from jax.experimental.pallas import tpu as pltpu
from jax.experimental.pallas import tpu_sc as plsc

EMBED = 64
BATCH = 16384
L = 16
NC, NS = 2, 16
NW = NC * NS
BPW = BATCH // NW
VOCAB_BLOCKS = 125000
NBUF = 8

_mesh = plsc.VectorSubcoreMesh(core_axis_name="c", subcore_axis_name="s")


@functools.partial(
    pl.kernel,
    out_type=jax.ShapeDtypeStruct((BATCH,), jnp.float32),
    mesh=_mesh,
    compiler_params=pltpu.CompilerParams(
        use_tc_tiling_on_sc=True, needs_layout_passes=False
    ),
    scratch_types=[
        pltpu.VMEM((BPW,), jnp.int32),
        pltpu.VMEM((BPW,), jnp.int32),
        pltpu.VMEM((NBUF, 8, EMBED), jnp.float32),
        pltpu.VMEM((NBUF, 8, EMBED), jnp.float32),
        pltpu.VMEM((BPW,), jnp.float32),
    ]
    + [pltpu.SemaphoreType.DMA] * NBUF
    + [pltpu.SemaphoreType.DMA] * NBUF,
)
def _skipgram_sc(cw_hbm, tw_hbm, ct_hbm, tt_hbm, out_hbm,
                 cidx, tidx, cr, tr, outv, *sems):
    csems = sems[:NBUF]
    tsems = sems[NBUF:]
    wid = lax.axis_index("s") * NC + lax.axis_index("c")
    base = wid * BPW

    pltpu.sync_copy(cw_hbm.at[pl.ds(base, BPW)], cidx)
    pltpu.sync_copy(tw_hbm.at[pl.ds(base, BPW)], tidx)

    def fire(c_row, t_row, b):
        pltpu.async_copy(ct_hbm.at[c_row >> 3], cr.at[b], csems[b])
        pltpu.async_copy(tt_hbm.at[t_row >> 3], tr.at[b], tsems[b])

    def drain(c_row, t_row, b):
        pltpu.make_async_copy(ct_hbm.at[c_row >> 3], cr.at[b],
                              csems[b]).wait()
        pltpu.make_async_copy(tt_hbm.at[t_row >> 3], tr.at[b],
                              tsems[b]).wait()

    lane = lax.iota(jnp.int32, L)

    civ0 = cidx[pl.ds(0, L)]
    tiv0 = tidx[pl.ds(0, L)]
    for b in range(NBUF):
        fire(civ0[b], tiv0[b], b)


    def compute(c_row, t_row, b, bi, acc):
        sc = c_row & 7
        st = t_row & 7
        p = jnp.zeros((L,), jnp.float32)
        for c in range(EMBED // L):
            p = p + (cr[b, sc, pl.ds(c * L, L)]
                     * tr[b, st, pl.ds(c * L, L)])
        dot = jnp.sum(p)
        return jnp.where(lane == bi, dot, acc)

    def block_body(g, carry):
        civ = cidx[pl.ds(g * L, L)]
        tiv = tidx[pl.ds(g * L, L)]
        nciv = cidx[pl.ds((g + 1) * L, L)]
        ntiv = tidx[pl.ds((g + 1) * L, L)]
        acc = jnp.zeros((L,), jnp.float32)
        for b in range(NBUF):
            drain(civ[b], tiv[b], b)
            acc = compute(civ[b], tiv[b], b, b, acc)
            fire(civ[b + NBUF], tiv[b + NBUF], b)
        for b in range(NBUF, L):
            drain(civ[b], tiv[b], b - NBUF)
            acc = compute(civ[b], tiv[b], b - NBUF, b, acc)
            fire(nciv[b - NBUF], ntiv[b - NBUF], b - NBUF)
        outv[pl.ds(g * L, L)] = acc
        return carry

    n_blocks = BPW // L
    lax.fori_loop(0, n_blocks - 1, block_body, 0)

    g_last = n_blocks - 1
    civ = cidx[pl.ds(g_last * L, L)]
    tiv = tidx[pl.ds(g_last * L, L)]
    acc = jnp.zeros((L,), jnp.float32)
    for b in range(NBUF):
        drain(civ[b], tiv[b], b)
        acc = compute(civ[b], tiv[b], b, b, acc)
        fire(civ[b + NBUF], tiv[b + NBUF], b)
    for b in range(NBUF, L):
        drain(civ[b], tiv[b], b - NBUF)
        acc = compute(civ[b], tiv[b], b - NBUF, b, acc)
    outv[pl.ds(g_last * L, L)] = acc

    pltpu.sync_copy(outv, out_hbm.at[pl.ds(base, BPW)])


def kernel(center_words, target_words, center_table, target_table):
    return _skipgram_sc(
        center_words.astype(jnp.int32),
        target_words.astype(jnp.int32),
        center_table.reshape(VOCAB_BLOCKS, 8, EMBED),
        target_table.reshape(VOCAB_BLOCKS, 8, EMBED),
    )


# speedup vs baseline: 2.2013x; 1.0661x over previous
import functools

import jax
import jax.numpy as jnp
from jax import lax
from jax.experimental import pallas as pl
from jax.experimental.pallas import tpu as pltpu
from jax.experimental.pallas import tpu_sc as plsc

EMBED = 64
BATCH = 16384
L = 16
NC, NS = 2, 16
NW = NC * NS
BPW = BATCH // NW
VOCAB_BLOCKS = 125000
NBUF = 8

_mesh = plsc.VectorSubcoreMesh(core_axis_name="c", subcore_axis_name="s")


@functools.partial(
    pl.kernel,
    out_type=jax.ShapeDtypeStruct((BATCH,), jnp.float32),
    mesh=_mesh,
    compiler_params=pltpu.CompilerParams(
        use_tc_tiling_on_sc=True, needs_layout_passes=False
    ),
    scratch_types=[
        pltpu.VMEM((BPW,), jnp.int32),
        pltpu.VMEM((BPW,), jnp.int32),
        pltpu.VMEM((NBUF, EMBED), jnp.float32),
        pltpu.VMEM((NBUF, EMBED), jnp.float32),
        pltpu.VMEM((BPW,), jnp.float32),
    ]
    + [pltpu.SemaphoreType.DMA] * NBUF
    + [pltpu.SemaphoreType.DMA] * NBUF,
)
def _skipgram_sc(cw_hbm, tw_hbm, ct_hbm, tt_hbm, out_hbm,
                 cidx, tidx, cr, tr, outv, *sems):
    csems = sems[:NBUF]
    tsems = sems[NBUF:]
    wid = lax.axis_index("s") * NC + lax.axis_index("c")
    base = wid * BPW

    pltpu.sync_copy(cw_hbm.at[pl.ds(base, BPW)], cidx)
    pltpu.sync_copy(tw_hbm.at[pl.ds(base, BPW)], tidx)

    def fire(c_row, t_row, b):
        pltpu.async_copy(ct_hbm.at[c_row >> 3, c_row & 7], cr.at[b], csems[b])
        pltpu.async_copy(tt_hbm.at[t_row >> 3, t_row & 7], tr.at[b], tsems[b])

    def drain(c_row, t_row, b):
        pltpu.make_async_copy(ct_hbm.at[c_row >> 3, c_row & 7], cr.at[b],
                              csems[b]).wait()
        pltpu.make_async_copy(tt_hbm.at[t_row >> 3, t_row & 7], tr.at[b],
                              tsems[b]).wait()

    lane = lax.iota(jnp.int32, L)

    civ0 = cidx[pl.ds(0, L)]
    tiv0 = tidx[pl.ds(0, L)]
    for b in range(NBUF):
        fire(civ0[b], tiv0[b], b)


    def compute(c_row, t_row, b, bi, acc):
        p = jnp.zeros((L,), jnp.float32)
        for c in range(EMBED // L):
            p = p + (cr[b, pl.ds(c * L, L)]
                     * tr[b, pl.ds(c * L, L)])
        dot = jnp.sum(p)
        return jnp.where(lane == bi, dot, acc)

    def block_body(g, carry):
        civ = cidx[pl.ds(g * L, L)]
        tiv = tidx[pl.ds(g * L, L)]
        nciv = cidx[pl.ds((g + 1) * L, L)]
        ntiv = tidx[pl.ds((g + 1) * L, L)]
        acc = jnp.zeros((L,), jnp.float32)
        for b in range(NBUF):
            drain(civ[b], tiv[b], b)
            acc = compute(civ[b], tiv[b], b, b, acc)
            fire(civ[b + NBUF], tiv[b + NBUF], b)
        for b in range(NBUF, L):
            drain(civ[b], tiv[b], b - NBUF)
            acc = compute(civ[b], tiv[b], b - NBUF, b, acc)
            fire(nciv[b - NBUF], ntiv[b - NBUF], b - NBUF)
        outv[pl.ds(g * L, L)] = acc
        return carry

    n_blocks = BPW // L
    lax.fori_loop(0, n_blocks - 1, block_body, 0)

    g_last = n_blocks - 1
    civ = cidx[pl.ds(g_last * L, L)]
    tiv = tidx[pl.ds(g_last * L, L)]
    acc = jnp.zeros((L,), jnp.float32)
    for b in range(NBUF):
        drain(civ[b], tiv[b], b)
        acc = compute(civ[b], tiv[b], b, b, acc)
        fire(civ[b + NBUF], tiv[b + NBUF], b)
    for b in range(NBUF, L):
        drain(civ[b], tiv[b], b - NBUF)
        acc = compute(civ[b], tiv[b], b - NBUF, b, acc)
    outv[pl.ds(g_last * L, L)] = acc

    pltpu.sync_copy(outv, out_hbm.at[pl.ds(base, BPW)])


def kernel(center_words, target_words, center_table, target_table):
    return _skipgram_sc(
        center_words.astype(jnp.int32),
        target_words.astype(jnp.int32),
        center_table.reshape(VOCAB_BLOCKS, 8, EMBED),
        target_table.reshape(VOCAB_BLOCKS, 8, EMBED),
    )
